# baseline (speedup 1.0000x reference)
import functools

import jax
import jax.numpy as jnp
from jax import lax
from jax.experimental import pallas as pl
from jax.experimental.pallas import tpu as pltpu
from jax.experimental.pallas import tpu_sc as plsc

VOCAB = 100000
D_MODEL = 64
BATCH = 1024

_NC = 2
_NS = 16
_NW = _NC * _NS
_B_PER_W = BATCH // _NW


def _sc_gather(emb_table, indices):
    mesh = plsc.VectorSubcoreMesh(core_axis_name="c", subcore_axis_name="s")

    @functools.partial(
        pl.kernel,
        mesh=mesh,
        out_type=jax.ShapeDtypeStruct((BATCH, D_MODEL), jnp.float32),
        scratch_types=[
            pltpu.VMEM((_B_PER_W,), jnp.int32),
            pltpu.VMEM((_B_PER_W, D_MODEL), jnp.float32),
            pltpu.SemaphoreType.DMA,
        ],
        compiler_params=pltpu.CompilerParams(use_tc_tiling_on_sc=False),
    )
    def gather_kernel(table_hbm, idx_hbm, out_hbm, idx_v, rows_v, sem):
        wid = lax.axis_index("s") * _NC + lax.axis_index("c")
        base = wid * _B_PER_W
        pltpu.sync_copy(idx_hbm.at[pl.ds(base, _B_PER_W)], idx_v)
        pltpu.async_copy(table_hbm.at[idx_v], rows_v, sem).wait()
        pltpu.sync_copy(rows_v, out_hbm.at[pl.ds(base, _B_PER_W)])

    return gather_kernel(emb_table, indices)


_TV = 2048


def _matmul_body(x_ref, w_ref, o_ref):
    o_ref[...] = lax.dot_general(
        x_ref[...],
        w_ref[...].astype(jnp.bfloat16),
        dimension_numbers=(((1,), (1,)), ((), ())),
        preferred_element_type=jnp.float32,
    )


def _tc_project(gathered, lin_weight):
    return pl.pallas_call(
        _matmul_body,
        grid=(pl.cdiv(VOCAB, _TV),),
        in_specs=[
            pl.BlockSpec((BATCH, D_MODEL), lambda i: (0, 0)),
            pl.BlockSpec((_TV, D_MODEL), lambda i: (i, 0)),
        ],
        out_specs=pl.BlockSpec((BATCH, _TV), lambda i: (0, i)),
        out_shape=jax.ShapeDtypeStruct((BATCH, VOCAB), jnp.float32),
        compiler_params=pltpu.CompilerParams(
            dimension_semantics=("parallel",),
        ),
    )(gathered, lin_weight)


@jax.jit
def kernel(indices, emb_table, lin_weight):
    gathered = _sc_gather(emb_table, indices.astype(jnp.int32))
    return _tc_project(gathered.astype(jnp.bfloat16), lin_weight)


# speedup vs baseline: 1.0076x; 1.0076x over previous
import functools

import jax
import jax.numpy as jnp
from jax import lax
from jax.experimental import pallas as pl
from jax.experimental.pallas import tpu as pltpu
from jax.experimental.pallas import tpu_sc as plsc

VOCAB = 100000
D_MODEL = 64
BATCH = 1024

_NC = 2
_NS = 16
_NW = _NC * _NS
_B_PER_W = BATCH // _NW


def _sc_gather(emb_table, indices):
    mesh = plsc.VectorSubcoreMesh(core_axis_name="c", subcore_axis_name="s")

    @functools.partial(
        pl.kernel,
        mesh=mesh,
        out_type=jax.ShapeDtypeStruct((BATCH, D_MODEL), jnp.float32),
        scratch_types=[
            pltpu.VMEM((_B_PER_W,), jnp.int32),
            pltpu.VMEM((_B_PER_W, D_MODEL), jnp.float32),
            pltpu.SemaphoreType.DMA,
        ],
        compiler_params=pltpu.CompilerParams(use_tc_tiling_on_sc=False),
    )
    def gather_kernel(table_hbm, idx_hbm, out_hbm, idx_v, rows_v, sem):
        wid = lax.axis_index("s") * _NC + lax.axis_index("c")
        base = wid * _B_PER_W
        pltpu.sync_copy(idx_hbm.at[pl.ds(base, _B_PER_W)], idx_v)
        pltpu.async_copy(table_hbm.at[idx_v], rows_v, sem).wait()
        pltpu.sync_copy(rows_v, out_hbm.at[pl.ds(base, _B_PER_W)])

    return gather_kernel(emb_table, indices)


_SUB = 1024
_NSUB = 4
_TV = _SUB * _NSUB
_NSTEP = pl.cdiv(VOCAB, _TV)
_TAIL = VOCAB - (_NSTEP - 1) * _TV - _SUB


def _matmul_body(x_ref, w_ref, out_ref, obuf, tbuf, sems, tsem):
    i = pl.program_id(0)
    last = _NSTEP - 1

    x = x_ref[...]

    for j in range(_NSUB):
        @pl.when(i > 0)
        def _():
            pltpu.make_async_copy(
                obuf.at[j],
                out_ref.at[:, pl.ds((i - 1) * _TV + j * _SUB, _SUB)],
                sems.at[j],
            ).wait()

        obuf[j] = lax.dot_general(
            x,
            w_ref[j * _SUB:(j + 1) * _SUB, :].astype(jnp.bfloat16),
            dimension_numbers=(((1,), (1,)), ((), ())),
            preferred_element_type=jnp.float32,
        )

        @pl.when(jnp.logical_or(i < last, j == 0))
        def _():
            pltpu.make_async_copy(
                obuf.at[j],
                out_ref.at[:, pl.ds(i * _TV + j * _SUB, _SUB)],
                sems.at[j],
            ).start()

    @pl.when(i == last)
    def _():
        tbuf[...] = lax.dot_general(
            x,
            w_ref[_SUB:_SUB + _TAIL, :].astype(jnp.bfloat16),
            dimension_numbers=(((1,), (1,)), ((), ())),
            preferred_element_type=jnp.float32,
        )
        tail_start = last * _TV + _SUB
        pltpu.make_async_copy(
            tbuf, out_ref.at[:, pl.ds(tail_start, _TAIL)], tsem
        ).start()
        pltpu.make_async_copy(
            obuf.at[0],
            out_ref.at[:, pl.ds(last * _TV, _SUB)],
            sems.at[0],
        ).wait()
        pltpu.make_async_copy(
            tbuf, out_ref.at[:, pl.ds(tail_start, _TAIL)], tsem
        ).wait()


def _tc_project(gathered_bf16, lin_weight):
    return pl.pallas_call(
        _matmul_body,
        grid=(_NSTEP,),
        in_specs=[
            pl.BlockSpec((BATCH, D_MODEL), lambda i: (0, 0)),
            pl.BlockSpec((_TV, D_MODEL), lambda i: (i, 0)),
        ],
        out_specs=pl.BlockSpec(memory_space=pl.ANY),
        out_shape=jax.ShapeDtypeStruct((BATCH, VOCAB), jnp.float32),
        scratch_shapes=[
            pltpu.VMEM((_NSUB, BATCH, _SUB), jnp.float32),
            pltpu.VMEM((BATCH, _TAIL), jnp.float32),
            pltpu.SemaphoreType.DMA((_NSUB,)),
            pltpu.SemaphoreType.DMA,
        ],
        compiler_params=pltpu.CompilerParams(
            dimension_semantics=("arbitrary",),
            vmem_limit_bytes=100 * 1024 * 1024,
        ),
    )(gathered_bf16, lin_weight)


@jax.jit
def kernel(indices, emb_table, lin_weight):
    gathered = _sc_gather(emb_table, indices.astype(jnp.int32))
    return _tc_project(gathered.astype(jnp.bfloat16), lin_weight)
